# baseline (speedup 1.0000x reference)
import functools
import jax
import jax.numpy as jnp
from jax import lax
from jax.experimental import pallas as pl
from jax.experimental.pallas import tpu as pltpu
from jax.experimental.pallas import tpu_sc as plsc

_N_PATCHES = 16
_CH = 16


@functools.lru_cache(maxsize=None)
def _make_sc_kernel(R, D, S):
    info = plsc.get_sparse_core_info()
    NC, NS, L = info.num_cores, info.num_subcores, info.num_lanes
    NW = NC * NS
    rows_w = R // NW
    rpp = S // _N_PATCHES
    ppw = rows_w // rpp
    wpb = S // rows_w
    nchunks = rows_w // _CH
    cpp = rpp // _CH
    cols = D // L

    mesh = plsc.VectorSubcoreMesh(core_axis_name="c", subcore_axis_name="s")

    @functools.partial(
        pl.kernel,
        out_type=jax.ShapeDtypeStruct((R * D,), jnp.float32),
        mesh=mesh,
        scratch_types=[
            pltpu.VMEM((ppw * D,), jnp.float32),
            pltpu.VMEM((_CH * D,), jnp.float32),
        ],
    )
    def sc_k(vf_hbm, w_hbm, out_hbm, w_buf, buf):
        cid = lax.axis_index("c")
        sid = lax.axis_index("s")
        wid = sid * NC + cid
        row0 = wid * rows_w
        p0 = (wid % wpb) * ppw
        pltpu.sync_copy(w_hbm.at[pl.ds(p0 * D, ppw * D)], w_buf)

        def chunk(i, carry):
            base = (row0 + i * _CH) * D
            woff = (i // cpp) * D
            pltpu.sync_copy(vf_hbm.at[pl.ds(base, _CH * D)], buf)

            def col(c, cc):
                wv = w_buf[pl.ds(woff + c * L, L)]
                for r in range(_CH):
                    off = r * D + c * L
                    buf[pl.ds(off, L)] = buf[pl.ds(off, L)] + wv
                return cc

            lax.fori_loop(0, cols, col, 0)
            pltpu.sync_copy(buf, out_hbm.at[pl.ds(base, _CH * D)])
            return carry

        lax.fori_loop(0, nchunks, chunk, 0)

    return sc_k


def kernel(vision_features, W):
    B, S, D = vision_features.shape
    R = B * S
    vf = vision_features.reshape(R * D)
    w_flat = W.reshape(-1)
    sc_k = _make_sc_kernel(R, D, S)
    out = sc_k(vf, w_flat)
    return out.reshape(B, S, D)
